# baseline (speedup 1.0000x reference)
import functools

import jax
import jax.numpy as jnp
from jax import lax
from jax.experimental import pallas as pl
from jax.experimental.pallas import tpu as pltpu
from jax.experimental.pallas import tpu_sc as plsc


def _build(N, V, D, num_cores, num_subcores):
    NW = num_cores * num_subcores
    per_w = N // NW
    C = 512
    n_chunks = per_w // C
    mesh = plsc.VectorSubcoreMesh(core_axis_name="c", subcore_axis_name="s")

    @functools.partial(
        pl.kernel,
        mesh=mesh,
        out_type=jax.ShapeDtypeStruct((N, D), jnp.float32),
        scratch_types=[
            pltpu.VMEM((C,), jnp.int32),
            pltpu.VMEM((C, D), jnp.float32),
            pltpu.SemaphoreType.DMA,
        ],
        compiler_params=pltpu.CompilerParams(use_tc_tiling_on_sc=False),
    )
    def run(idx_hbm, table_hbm, out_hbm, idx_v, rows_v, sem):
        wid = lax.axis_index("s") * num_cores + lax.axis_index("c")
        base = wid * per_w

        def body(i, carry):
            off = pl.multiple_of(base + i * C, 8)
            pltpu.sync_copy(idx_hbm.at[pl.ds(off, C)], idx_v)
            pltpu.async_copy(table_hbm.at[idx_v], rows_v, sem).wait()
            pltpu.sync_copy(rows_v, out_hbm.at[pl.ds(off, C)])
            return carry

        lax.fori_loop(0, n_chunks, body, 0)

    return run


def kernel(x, table):
    B, H = x.shape
    V, D = table.shape
    N = B * H
    info = plsc.get_sparse_core_info()
    run = _build(N, V, D, info.num_cores, info.num_subcores)
    out = run(x.reshape(N).astype(jnp.int32), table)
    return out.reshape(B, H, D)


# speedup vs baseline: 1.0417x; 1.0417x over previous
import functools

import jax
import jax.numpy as jnp
from jax import lax
from jax.experimental import pallas as pl
from jax.experimental.pallas import tpu as pltpu
from jax.experimental.pallas import tpu_sc as plsc


def _build(N, V, D, num_cores, num_subcores):
    NW = num_cores * num_subcores
    per_w = N // NW
    C = 320
    NBUF = 4
    n_chunks = per_w // C
    n_rounds = n_chunks // NBUF
    assert per_w % C == 0 and n_chunks % NBUF == 0
    mesh = plsc.VectorSubcoreMesh(core_axis_name="c", subcore_axis_name="s")

    @functools.partial(
        pl.kernel,
        mesh=mesh,
        out_type=jax.ShapeDtypeStruct((N, D), jnp.float32),
        scratch_types=[
            pltpu.VMEM((per_w,), jnp.int32),
            pltpu.VMEM((NBUF, C, D), jnp.float32),
            pltpu.SemaphoreType.DMA((NBUF,)),
            pltpu.SemaphoreType.DMA((NBUF,)),
        ],
        compiler_params=pltpu.CompilerParams(use_tc_tiling_on_sc=False),
    )
    def run(idx_hbm, table_hbm, out_hbm, idx_v, rows_v, g_sem, o_sem):
        wid = lax.axis_index("s") * num_cores + lax.axis_index("c")
        base = pl.multiple_of(wid * per_w, 8)
        pltpu.sync_copy(idx_hbm.at[pl.ds(base, per_w)], idx_v)

        def g_copy(i, b):
            return pltpu.make_async_copy(
                table_hbm.at[idx_v.at[pl.ds(i * C, C)]],
                rows_v.at[b], g_sem.at[b])

        def o_copy(i, b):
            return pltpu.make_async_copy(
                rows_v.at[b], out_hbm.at[pl.ds(base + i * C, C)], o_sem.at[b])

        for b in range(NBUF):
            g_copy(b, b).start()

        def body(j, carry):
            i0 = j * NBUF
            for b in range(NBUF):
                g_copy(i0 + b, b).wait()
                o_copy(i0 + b, b).start()

            @pl.when(j + 1 < n_rounds)
            def _():
                for b in range(NBUF):
                    o_copy(i0 + b, b).wait()
                    g_copy(i0 + NBUF + b, b).start()

            return carry

        lax.fori_loop(0, n_rounds, body, 0)
        for b in range(NBUF):
            o_copy(n_chunks - NBUF + b, b).wait()

    return run


def kernel(x, table):
    B, H = x.shape
    V, D = table.shape
    N = B * H
    info = plsc.get_sparse_core_info()
    run = _build(N, V, D, info.num_cores, info.num_subcores)
    out = run(x.reshape(N).astype(jnp.int32), table)
    return out.reshape(B, H, D)
